# baseline (speedup 1.0000x reference)
import jax
import jax.numpy as jnp
from jax.experimental import pallas as pl
from jax.experimental.pallas import tpu as pltpu

_N = 2097152
_K = 209715
_ROWS = 2048
_COLS = 1024
_CHUNKS = 8
_CROWS = _ROWS // _CHUNKS
_HI_TOP = 0x42C8
_ITERS_A = 15
_ITERS_B = 16


def _body(p_ref, t_ref, out_ref, bce_ref, t16_ref, lo16_ref, acc_ref):
    i = pl.program_id(0)

    @pl.when(i == 0)
    def _init():
        acc_ref[0] = 0.0
        acc_ref[1] = 0.0
        acc_ref[2] = 0.0

    p = p_ref[...]
    t = t_ref[...]
    acc_ref[0] += jnp.sum(p)
    acc_ref[1] += jnp.sum(t)
    acc_ref[2] += jnp.sum(p * t)

    log_p = jnp.maximum(jnp.log(p), -100.0)
    log_1mp = jnp.maximum(jnp.log1p(-p), -100.0)
    bce = jnp.maximum(-(t * log_p + (1.0 - t) * log_1mp), 0.0)
    bits = pltpu.bitcast(bce, jnp.int32)
    rows = pl.ds(i * _CROWS, _CROWS)
    bce_ref[rows, :] = bce
    t16_ref[rows, :] = (bits >> 16).astype(jnp.int16)
    lo16_ref[rows, :] = ((bits & 0xFFFF) - 32768).astype(jnp.int16)

    @pl.when(i == _CHUNKS - 1)
    def _select():
        def step_a(_, carry):
            lo, hi = carry
            mid = lo + (hi - lo + 1) // 2
            c = jnp.sum((t16_ref[...] >= mid.astype(jnp.int16)).astype(jnp.int32))
            big = c >= _K
            return jnp.where(big, mid, lo), jnp.where(big, hi, mid - 1)

        top, _ = jax.lax.fori_loop(
            0, _ITERS_A, step_a, (jnp.int32(0), jnp.int32(_HI_TOP))
        )
        top16 = top.astype(jnp.int16)

        t16 = t16_ref[...]
        c_top = jnp.sum((t16 > top16).astype(jnp.int32))
        lo16_ref[...] = jnp.where(t16 == top16, lo16_ref[...], jnp.int16(-32768))

        def step_b(_, carry):
            lo, hi = carry
            mid = lo + (hi - lo + 1) // 2
            ms = (mid - 32768).astype(jnp.int16)
            c = c_top + jnp.sum((lo16_ref[...] >= ms).astype(jnp.int32))
            big = c >= _K
            return jnp.where(big, mid, lo), jnp.where(big, hi, mid - 1)

        low, _ = jax.lax.fori_loop(
            0, _ITERS_B, step_b, (jnp.int32(0), jnp.int32(0xFFFF))
        )
        low16 = (low - 32768).astype(jnp.int16)

        lo16m = lo16_ref[...]
        b = bce_ref[...]
        vk = jnp.max(jnp.where((t16 == top16) & (lo16m == low16), b, 0.0))
        gt = (t16 > top16) | ((t16 == top16) & (lo16m > low16))
        c_gt = jnp.sum(gt.astype(jnp.int32))
        s_gt = jnp.sum(jnp.where(gt, b, 0.0))
        topk_mean = (s_gt + (_K - c_gt).astype(jnp.float32) * vk) / _K
        dice = 1.0 - (2.0 * acc_ref[2] + 1.0) / (acc_ref[0] + acc_ref[1] + 1.0)
        out_ref[...] = (dice + topk_mean).reshape(1, 1)


def kernel(preds, gt_masks):
    p = preds.reshape(_ROWS, _COLS)
    t = gt_masks.reshape(_ROWS, _COLS)
    out = pl.pallas_call(
        _body,
        grid=(_CHUNKS,),
        in_specs=[
            pl.BlockSpec((_CROWS, _COLS), lambda i: (i, 0)),
            pl.BlockSpec((_CROWS, _COLS), lambda i: (i, 0)),
        ],
        out_specs=pl.BlockSpec((1, 1), lambda i: (0, 0)),
        out_shape=jax.ShapeDtypeStruct((1, 1), jnp.float32),
        scratch_shapes=[
            pltpu.VMEM((_ROWS, _COLS), jnp.float32),
            pltpu.VMEM((_ROWS, _COLS), jnp.int16),
            pltpu.VMEM((_ROWS, _COLS), jnp.int16),
            pltpu.SMEM((4,), jnp.float32),
        ],
    )(p, t)
    return out[0, 0]


# speedup vs baseline: 1.0718x; 1.0718x over previous
import jax
import jax.numpy as jnp
from jax.experimental import pallas as pl
from jax.experimental.pallas import tpu as pltpu

_N = 2097152
_K = 209715
_NMK = _N - _K
_ROWS = 2048
_COLS = 1024
_CHUNKS = 8
_CROWS = _ROWS // _CHUNKS
_HI_BITS = 0x42C80000
_SEARCH_ITERS = 31


def _body(p_ref, t_ref, out_ref, bits_ref, acc_ref):
    i = pl.program_id(0)

    @pl.when(i == 0)
    def _init():
        acc_ref[0] = 0.0
        acc_ref[1] = 0.0
        acc_ref[2] = 0.0

    p = p_ref[...]
    t = t_ref[...]
    acc_ref[0] += jnp.sum(p)
    acc_ref[1] += jnp.sum(t)
    acc_ref[2] += jnp.sum(p * t)

    log_p = jnp.maximum(jnp.log(p), -100.0)
    log_1mp = jnp.maximum(jnp.log1p(-p), -100.0)
    bce = jnp.maximum(-(t * log_p + (1.0 - t) * log_1mp), 0.0)
    bits_ref[pl.ds(i * _CROWS, _CROWS), :] = pltpu.bitcast(bce, jnp.int32)

    @pl.when(i == _CHUNKS - 1)
    def _select():
        def step(_, carry):
            lo, hi = carry
            mid = lo + (hi - lo + 1) // 2
            c_lt = jnp.sum(
                jax.lax.shift_right_logical(bits_ref[...] - mid, 31)
            )
            big = c_lt <= _NMK
            lo = jnp.where(big, mid, lo)
            hi = jnp.where(big, hi, mid - 1)
            return lo, hi

        lo, _ = jax.lax.fori_loop(
            0, _SEARCH_ITERS, step, (jnp.int32(0), jnp.int32(_HI_BITS))
        )

        bits = bits_ref[...]
        b = pltpu.bitcast(bits, jnp.float32)
        le = jax.lax.shift_right_logical(bits - (lo + 1), 31)
        c_gt = _N - jnp.sum(le)
        s_gt = jnp.sum(b * (1 - le).astype(jnp.float32))
        vk = jnp.max(jnp.where(bits == lo, b, 0.0))
        topk_mean = (s_gt + (_K - c_gt).astype(jnp.float32) * vk) / _K
        dice = 1.0 - (2.0 * acc_ref[2] + 1.0) / (acc_ref[0] + acc_ref[1] + 1.0)
        out_ref[...] = (dice + topk_mean).reshape(1, 1)


def kernel(preds, gt_masks):
    p = preds.reshape(_ROWS, _COLS)
    t = gt_masks.reshape(_ROWS, _COLS)
    out = pl.pallas_call(
        _body,
        grid=(_CHUNKS,),
        in_specs=[
            pl.BlockSpec((_CROWS, _COLS), lambda i: (i, 0)),
            pl.BlockSpec((_CROWS, _COLS), lambda i: (i, 0)),
        ],
        out_specs=pl.BlockSpec((1, 1), lambda i: (0, 0)),
        out_shape=jax.ShapeDtypeStruct((1, 1), jnp.float32),
        scratch_shapes=[
            pltpu.VMEM((_ROWS, _COLS), jnp.int32),
            pltpu.SMEM((4,), jnp.float32),
        ],
    )(p, t)
    return out[0, 0]


# speedup vs baseline: 1.6145x; 1.5063x over previous
import jax
import jax.numpy as jnp
from jax.experimental import pallas as pl
from jax.experimental.pallas import tpu as pltpu

_N = 2097152
_K = 209715
_NMK = _N - _K
_ROWS = 2048
_COLS = 1024
_CHUNKS = 8
_CROWS = _ROWS // _CHUNKS
_HI_BITS = 0x42C80000
_SEARCH_ITERS = 31


def _body(p_ref, t_ref, out_ref, bits_ref, acc_ref):
    i = pl.program_id(0)

    @pl.when(i == 0)
    def _init():
        acc_ref[0] = 0.0
        acc_ref[1] = 0.0
        acc_ref[2] = 0.0

    p = p_ref[...]
    t = t_ref[...]
    acc_ref[0] += jnp.sum(p)
    acc_ref[1] += jnp.sum(t)
    acc_ref[2] += jnp.sum(p * t)

    log_p = jnp.maximum(jnp.log(p), -100.0)
    log_1mp = jnp.maximum(jnp.log1p(-p), -100.0)
    bce = jnp.maximum(-(t * log_p + (1.0 - t) * log_1mp), 0.0)
    bits_ref[pl.ds(i * _CROWS, _CROWS), :] = pltpu.bitcast(bce, jnp.int32)

    @pl.when(i == _CHUNKS - 1)
    def _select():
        def step(_, carry):
            lo, hi = carry
            mid = lo + (hi - lo + 1) // 2
            parts = []
            for j in range(8):
                sl = bits_ref[pl.ds(j * (_ROWS // 8), _ROWS // 8), :]
                parts.append(
                    jnp.sum(jax.lax.shift_right_logical(sl - mid, 31))
                )
            c_lt = sum(parts)
            big = c_lt <= _NMK
            lo = jnp.where(big, mid, lo)
            hi = jnp.where(big, hi, mid - 1)
            return lo, hi

        lo, _ = jax.lax.fori_loop(
            0, _SEARCH_ITERS, step, (jnp.int32(0), jnp.int32(_HI_BITS))
        )

        c_le_parts, s_parts, vk_parts = [], [], []
        for j in range(8):
            rows = pl.ds(j * (_ROWS // 8), _ROWS // 8)
            bits = bits_ref[rows, :]
            b = pltpu.bitcast(bits, jnp.float32)
            le = jax.lax.shift_right_logical(bits - (lo + 1), 31)
            c_le_parts.append(jnp.sum(le))
            s_parts.append(jnp.sum(b * (1 - le).astype(jnp.float32)))
            vk_parts.append(jnp.max(jnp.where(bits == lo, b, 0.0)))
        c_gt = _N - sum(c_le_parts)
        s_gt = sum(s_parts)
        vk = jnp.max(jnp.stack(vk_parts))
        topk_mean = (s_gt + (_K - c_gt).astype(jnp.float32) * vk) / _K
        dice = 1.0 - (2.0 * acc_ref[2] + 1.0) / (acc_ref[0] + acc_ref[1] + 1.0)
        out_ref[...] = (dice + topk_mean).reshape(1, 1)


def kernel(preds, gt_masks):
    p = preds.reshape(_ROWS, _COLS)
    t = gt_masks.reshape(_ROWS, _COLS)
    out = pl.pallas_call(
        _body,
        grid=(_CHUNKS,),
        in_specs=[
            pl.BlockSpec((_CROWS, _COLS), lambda i: (i, 0)),
            pl.BlockSpec((_CROWS, _COLS), lambda i: (i, 0)),
        ],
        out_specs=pl.BlockSpec((1, 1), lambda i: (0, 0)),
        out_shape=jax.ShapeDtypeStruct((1, 1), jnp.float32),
        scratch_shapes=[
            pltpu.VMEM((_ROWS, _COLS), jnp.int32),
            pltpu.SMEM((4,), jnp.float32),
        ],
    )(p, t)
    return out[0, 0]


# speedup vs baseline: 1.8705x; 1.1586x over previous
import jax
import jax.numpy as jnp
from jax.experimental import pallas as pl
from jax.experimental.pallas import tpu as pltpu

_N = 2097152
_K = 209715
_NMK = _N - _K
_ROWS = 2048
_COLS = 1024
_CHUNKS = 8
_CROWS = _ROWS // _CHUNKS
_HI_BITS = 0x42C80000
_SEARCH_ITERS = 22
_SLICES = 8
_SROWS = _ROWS // _SLICES


def _body(p_ref, t_ref, out_ref, bits_ref, acc_ref):
    i = pl.program_id(0)

    @pl.when(i == 0)
    def _init():
        acc_ref[0] = 0.0
        acc_ref[1] = 0.0
        acc_ref[2] = 0.0

    sp, st, si = [], [], []
    for j in range(4):
        rows = pl.ds(j * (_CROWS // 4), _CROWS // 4)
        p = p_ref[rows, :]
        t = t_ref[rows, :]
        sp.append(jnp.sum(p))
        st.append(jnp.sum(t))
        si.append(jnp.sum(p * t))
    acc_ref[0] += sum(sp)
    acc_ref[1] += sum(st)
    acc_ref[2] += sum(si)

    p = p_ref[...]
    t = t_ref[...]
    log_p = jnp.maximum(jnp.log(p), -100.0)
    log_1mp = jnp.maximum(jnp.log1p(-p), -100.0)
    bce = jnp.maximum(-(t * log_p + (1.0 - t) * log_1mp), 0.0)
    bits_ref[pl.ds(i * _CROWS, _CROWS), :] = pltpu.bitcast(bce, jnp.int32)

    @pl.when(i == _CHUNKS - 1)
    def _select():
        def step(_, carry):
            lo, hi = carry
            mid = lo + (hi - lo + 1) // 2
            parts = []
            for j in range(_SLICES):
                sl = bits_ref[pl.ds(j * _SROWS, _SROWS), :]
                parts.append(
                    jnp.sum(jax.lax.shift_right_logical(sl - mid, 31))
                )
            c_lt = sum(parts)
            big = c_lt <= _NMK
            lo = jnp.where(big, mid, lo)
            hi = jnp.where(big, hi, mid - 1)
            return lo, hi

        lo, hi = jax.lax.fori_loop(
            0, _SEARCH_ITERS, step, (jnp.int32(0), jnp.int32(_HI_BITS))
        )

        c_le_parts, s_parts, vk_parts = [], [], []
        for j in range(_SLICES):
            rows = pl.ds(j * _SROWS, _SROWS)
            bits = bits_ref[rows, :]
            b = pltpu.bitcast(bits, jnp.float32)
            le = jax.lax.shift_right_logical(bits - (lo + 1), 31)
            c_le_parts.append(jnp.sum(le))
            s_parts.append(jnp.sum(b * (1 - le).astype(jnp.float32)))
            le_hi = jax.lax.shift_right_logical(bits - (hi + 1), 31)
            vk_parts.append(jnp.max(b * le_hi.astype(jnp.float32)))
        c_gt = _N - sum(c_le_parts)
        s_gt = sum(s_parts)
        vk = jnp.max(jnp.stack(vk_parts))

        topk_mean = (s_gt + (_K - c_gt).astype(jnp.float32) * vk) / _K
        dice = 1.0 - (2.0 * acc_ref[2] + 1.0) / (acc_ref[0] + acc_ref[1] + 1.0)
        out_ref[...] = (dice + topk_mean).reshape(1, 1)


def kernel(preds, gt_masks):
    p = preds.reshape(_ROWS, _COLS)
    t = gt_masks.reshape(_ROWS, _COLS)
    out = pl.pallas_call(
        _body,
        grid=(_CHUNKS,),
        in_specs=[
            pl.BlockSpec((_CROWS, _COLS), lambda i: (i, 0)),
            pl.BlockSpec((_CROWS, _COLS), lambda i: (i, 0)),
        ],
        out_specs=pl.BlockSpec((1, 1), lambda i: (0, 0)),
        out_shape=jax.ShapeDtypeStruct((1, 1), jnp.float32),
        scratch_shapes=[
            pltpu.VMEM((_ROWS, _COLS), jnp.int32),
            pltpu.SMEM((4,), jnp.float32),
        ],
    )(p, t)
    return out[0, 0]


# speedup vs baseline: 2.0400x; 1.0906x over previous
import jax
import jax.numpy as jnp
from jax.experimental import pallas as pl
from jax.experimental.pallas import tpu as pltpu

_N = 2097152
_K = 209715
_NMK = _N - _K
_ROWS = 2048
_COLS = 1024
_CHUNKS = 8
_CROWS = _ROWS // _CHUNKS
_HI_BITS = 0x42C80000
_SEARCH_ITERS = 22
_SLICES = 16
_SROWS = _ROWS // _SLICES


def _body(p_ref, t_ref, out_ref, bits_ref, acc_ref):
    i = pl.program_id(0)

    @pl.when(i == 0)
    def _init():
        acc_ref[0] = 0.0
        acc_ref[1] = 0.0
        acc_ref[2] = 0.0

    sp, st, si = [], [], []
    for j in range(4):
        rows = pl.ds(j * (_CROWS // 4), _CROWS // 4)
        p = p_ref[rows, :]
        t = t_ref[rows, :]
        pt = p * t
        sp.append(jnp.sum(p))
        st.append(jnp.sum(t))
        si.append(jnp.sum(pt))
        q = (1.0 - p) - t + 2.0 * pt
        bce = jnp.maximum(jnp.minimum(-jnp.log(q), 100.0), 0.0)
        bits_ref[pl.ds(i * _CROWS + j * (_CROWS // 4), _CROWS // 4), :] = (
            pltpu.bitcast(bce, jnp.int32)
        )
    acc_ref[0] += sum(sp)
    acc_ref[1] += sum(st)
    acc_ref[2] += sum(si)

    @pl.when(i == _CHUNKS - 1)
    def _select():
        def step(_, carry):
            lo, hi = carry
            mid = lo + (hi - lo + 1) // 2
            parts = []
            for j in range(_SLICES):
                sl = bits_ref[pl.ds(j * _SROWS, _SROWS), :]
                parts.append(
                    jnp.sum(jax.lax.shift_right_logical(sl - mid, 31))
                )
            c_lt = sum(parts)
            big = c_lt <= _NMK
            lo = jnp.where(big, mid, lo)
            hi = jnp.where(big, hi, mid - 1)
            return lo, hi

        lo, hi = jax.lax.fori_loop(
            0, _SEARCH_ITERS, step, (jnp.int32(0), jnp.int32(_HI_BITS))
        )

        c_le_parts, s_parts, vk_parts = [], [], []
        for j in range(_SLICES):
            rows = pl.ds(j * _SROWS, _SROWS)
            bits = bits_ref[rows, :]
            b = pltpu.bitcast(bits, jnp.float32)
            le = jax.lax.shift_right_logical(bits - (lo + 1), 31)
            c_le_parts.append(jnp.sum(le))
            s_parts.append(jnp.sum(b * (1 - le).astype(jnp.float32)))
            le_hi = jax.lax.shift_right_logical(bits - (hi + 1), 31)
            vk_parts.append(jnp.max(b * le_hi.astype(jnp.float32)))
        c_gt = _N - sum(c_le_parts)
        s_gt = sum(s_parts)
        vk = jnp.max(jnp.stack(vk_parts))

        topk_mean = (s_gt + (_K - c_gt).astype(jnp.float32) * vk) / _K
        dice = 1.0 - (2.0 * acc_ref[2] + 1.0) / (acc_ref[0] + acc_ref[1] + 1.0)
        out_ref[...] = (dice + topk_mean).reshape(1, 1)


def kernel(preds, gt_masks):
    p = preds.reshape(_ROWS, _COLS)
    t = gt_masks.reshape(_ROWS, _COLS)
    out = pl.pallas_call(
        _body,
        grid=(_CHUNKS,),
        in_specs=[
            pl.BlockSpec((_CROWS, _COLS), lambda i: (i, 0)),
            pl.BlockSpec((_CROWS, _COLS), lambda i: (i, 0)),
        ],
        out_specs=pl.BlockSpec((1, 1), lambda i: (0, 0)),
        out_shape=jax.ShapeDtypeStruct((1, 1), jnp.float32),
        scratch_shapes=[
            pltpu.VMEM((_ROWS, _COLS), jnp.int32),
            pltpu.SMEM((4,), jnp.float32),
        ],
    )(p, t)
    return out[0, 0]
